# baseline (speedup 1.0000x reference)
import functools

import jax
import jax.numpy as jnp
from jax import lax
from jax.experimental import pallas as pl
from jax.experimental.pallas import tpu as pltpu

C = 8
B = 4096
D = 768
BB = 128

_DNT = (((1,), (1,)), ((), ()))


def _cell_kernel(cm_ref, ch_ref, wf_ref, bf_ref, wiou_ref, biou_ref,
                 nm_ref, nh_ref):
    h = ch_ref[...]
    hs = jnp.sum(h, axis=0)

    iou = lax.dot_general(hs.astype(jnp.bfloat16),
                          wiou_ref[...].astype(jnp.bfloat16), _DNT,
                          preferred_element_type=jnp.float32) + biou_ref[...]
    input_gate = jax.nn.sigmoid(iou[:, :D])
    output_gate = jax.nn.sigmoid(iou[:, D:2 * D])
    memory_gate = jnp.tanh(iou[:, 2 * D:])

    h2 = h.reshape(C * BB, D).astype(jnp.bfloat16)
    f_logits = lax.dot_general(h2, wf_ref[...].astype(jnp.bfloat16), _DNT,
                               preferred_element_type=jnp.float32) + bf_ref[...]
    fmem = jax.nn.sigmoid(f_logits) * cm_ref[...].reshape(C * BB, D)
    fsum = jnp.sum(fmem.reshape(C, BB, D), axis=0)

    nm = input_gate * memory_gate + fsum
    nm_ref[...] = nm
    nh_ref[...] = output_gate * jnp.tanh(nm)


@functools.partial(jax.jit, static_argnames=("interpret",))
def kernel(children_memory, children_hidden, Wf, bf, Wiou, biou,
           interpret=False):
    bf2 = bf.reshape(1, D)
    biou2 = biou.reshape(1, 3 * D)

    grid = (B // BB,)
    nm, nh = pl.pallas_call(
        _cell_kernel,
        grid=grid,
        in_specs=[
            pl.BlockSpec((C, BB, D), lambda i: (0, i, 0)),
            pl.BlockSpec((C, BB, D), lambda i: (0, i, 0)),
            pl.BlockSpec((D, D), lambda i: (0, 0)),
            pl.BlockSpec((1, D), lambda i: (0, 0)),
            pl.BlockSpec((3 * D, D), lambda i: (0, 0)),
            pl.BlockSpec((1, 3 * D), lambda i: (0, 0)),
        ],
        out_specs=[
            pl.BlockSpec((BB, D), lambda i: (i, 0)),
            pl.BlockSpec((BB, D), lambda i: (i, 0)),
        ],
        out_shape=[
            jax.ShapeDtypeStruct((B, D), jnp.float32),
            jax.ShapeDtypeStruct((B, D), jnp.float32),
        ],
        compiler_params=pltpu.CompilerParams(
            dimension_semantics=("parallel",),
        ),
        interpret=interpret,
    )(children_memory, children_hidden, Wf, bf2, Wiou, biou2)
    return (nm, nh)


# speedup vs baseline: 1.2978x; 1.2978x over previous
import functools

import jax
import jax.numpy as jnp
from jax import lax
from jax.experimental import pallas as pl
from jax.experimental.pallas import tpu as pltpu

C = 8
B = 4096
D = 768
BB = 256

_DNT = (((1,), (1,)), ((), ()))


def _cell_kernel(cm_ref, ch_ref, wf_ref, bf_ref, wiou_ref, biou_ref,
                 nm_ref, nh_ref):
    wf = wf_ref[...].astype(jnp.bfloat16)
    bfv = bf_ref[...]
    hs = None
    fsum = None
    for c in range(C):
        hc = ch_ref[c]
        hs = hc if hs is None else hs + hc
        fl = lax.dot_general(hc.astype(jnp.bfloat16), wf, _DNT,
                             preferred_element_type=jnp.float32) + bfv
        fm = jax.nn.sigmoid(fl) * cm_ref[c]
        fsum = fm if fsum is None else fsum + fm

    iou = lax.dot_general(hs.astype(jnp.bfloat16),
                          wiou_ref[...].astype(jnp.bfloat16), _DNT,
                          preferred_element_type=jnp.float32) + biou_ref[...]
    input_gate = jax.nn.sigmoid(iou[:, :D])
    output_gate = jax.nn.sigmoid(iou[:, D:2 * D])
    memory_gate = jnp.tanh(iou[:, 2 * D:])

    nm = input_gate * memory_gate + fsum
    nm_ref[...] = nm
    nh_ref[...] = output_gate * jnp.tanh(nm)


@functools.partial(jax.jit, static_argnames=("interpret",))
def kernel(children_memory, children_hidden, Wf, bf, Wiou, biou,
           interpret=False):
    bf2 = bf.reshape(1, D)
    biou2 = biou.reshape(1, 3 * D)

    grid = (B // BB,)
    nm, nh = pl.pallas_call(
        _cell_kernel,
        grid=grid,
        in_specs=[
            pl.BlockSpec((C, BB, D), lambda i: (0, i, 0)),
            pl.BlockSpec((C, BB, D), lambda i: (0, i, 0)),
            pl.BlockSpec((D, D), lambda i: (0, 0)),
            pl.BlockSpec((1, D), lambda i: (0, 0)),
            pl.BlockSpec((3 * D, D), lambda i: (0, 0)),
            pl.BlockSpec((1, 3 * D), lambda i: (0, 0)),
        ],
        out_specs=[
            pl.BlockSpec((BB, D), lambda i: (i, 0)),
            pl.BlockSpec((BB, D), lambda i: (i, 0)),
        ],
        out_shape=[
            jax.ShapeDtypeStruct((B, D), jnp.float32),
            jax.ShapeDtypeStruct((B, D), jnp.float32),
        ],
        compiler_params=pltpu.CompilerParams(
            dimension_semantics=("parallel",),
        ),
        interpret=interpret,
    )(children_memory, children_hidden, Wf, bf2, Wiou, biou2)
    return (nm, nh)


# speedup vs baseline: 1.3011x; 1.0026x over previous
import functools

import jax
import jax.numpy as jnp
from jax import lax
from jax.experimental import pallas as pl
from jax.experimental.pallas import tpu as pltpu

C = 8
B = 4096
D = 768
BB = 256

_DNT = (((1,), (1,)), ((), ()))


def _sigmoid(x):
    return 0.5 * jnp.tanh(0.5 * x) + 0.5


def _cell_kernel(cm_ref, ch_ref, wf_ref, bf_ref, wiou_ref, biou_ref,
                 nm_ref, nh_ref):
    wf = wf_ref[...].astype(jnp.bfloat16)
    bfv = bf_ref[...]
    hs = None
    fsum = None
    for c in range(C):
        hc = ch_ref[c]
        hs = hc if hs is None else hs + hc
        fl = lax.dot_general(hc.astype(jnp.bfloat16), wf, _DNT,
                             preferred_element_type=jnp.float32) + bfv
        fm = _sigmoid(fl) * cm_ref[c]
        fsum = fm if fsum is None else fsum + fm

    iou = lax.dot_general(hs.astype(jnp.bfloat16),
                          wiou_ref[...].astype(jnp.bfloat16), _DNT,
                          preferred_element_type=jnp.float32) + biou_ref[...]
    input_gate = _sigmoid(iou[:, :D])
    output_gate = _sigmoid(iou[:, D:2 * D])
    memory_gate = jnp.tanh(iou[:, 2 * D:])

    nm = input_gate * memory_gate + fsum
    nm_ref[...] = nm
    nh_ref[...] = output_gate * jnp.tanh(nm)


@functools.partial(jax.jit, static_argnames=("interpret",))
def kernel(children_memory, children_hidden, Wf, bf, Wiou, biou,
           interpret=False):
    bf2 = bf.reshape(1, D)
    biou2 = biou.reshape(1, 3 * D)

    grid = (B // BB,)
    nm, nh = pl.pallas_call(
        _cell_kernel,
        grid=grid,
        in_specs=[
            pl.BlockSpec((C, BB, D), lambda i: (0, i, 0)),
            pl.BlockSpec((C, BB, D), lambda i: (0, i, 0)),
            pl.BlockSpec((D, D), lambda i: (0, 0)),
            pl.BlockSpec((1, D), lambda i: (0, 0)),
            pl.BlockSpec((3 * D, D), lambda i: (0, 0)),
            pl.BlockSpec((1, 3 * D), lambda i: (0, 0)),
        ],
        out_specs=[
            pl.BlockSpec((BB, D), lambda i: (i, 0)),
            pl.BlockSpec((BB, D), lambda i: (i, 0)),
        ],
        out_shape=[
            jax.ShapeDtypeStruct((B, D), jnp.float32),
            jax.ShapeDtypeStruct((B, D), jnp.float32),
        ],
        compiler_params=pltpu.CompilerParams(
            dimension_semantics=("parallel",),
        ),
        interpret=interpret,
    )(children_memory, children_hidden, Wf, bf2, Wiou, biou2)
    return (nm, nh)
